# baseline (speedup 1.0000x reference)
import functools

import jax
import jax.numpy as jnp
from jax import lax
from jax.experimental import pallas as pl
from jax.experimental.pallas import tpu as pltpu
from jax.experimental.pallas import tpu_sc as plsc

EPS = 1e-06

_PERMS = (
    (2, 1, 0),
    (1, 2, 0),
    (0, 1, 2),
    (1, 0, 2),
    (2, 0, 1),
    (0, 1, 2),
    (0, 2, 1),
    (0, 1, 2),
)

_L = 16
_NC = 2
_NS = 16
_NW = _NC * _NS
_DQ = 33
_ROW = 450
_CHUNK = 128


def _round_to_bf16(x):
    u = lax.bitcast_convert_type(x, jnp.int32)
    lsb = lax.shift_right_logical(u, 16) & jnp.int32(1)
    r = (u + jnp.int32(0x7FFF) + lsb) & jnp.int32(-65536)
    return lax.bitcast_convert_type(r, jnp.float32)


def _sc_body(qt_hbm, ksplat_hbm, ikn_hbm, table_hbm, out_hbm,
             qv, kv, iknv, tabv, idxv, chunkbuf, qw, nch):
    wid = lax.axis_index("s") * _NC + lax.axis_index("c")
    qbase = wid * qw

    pltpu.sync_copy(qt_hbm.at[:, pl.ds(qbase, qw)], qv)
    pltpu.sync_copy(ksplat_hbm, kv)
    pltpu.sync_copy(ikn_hbm, iknv)
    pltpu.sync_copy(table_hbm, tabv)

    def group(g, carry):
        acc0 = jnp.zeros((_L,), jnp.float32)
        acc1 = jnp.zeros((_L,), jnp.float32)
        acc2 = jnp.zeros((_L,), jnp.float32)
        for d in range(_DQ):
            qvec = qv[d, pl.ds(g * _L, _L)]
            acc0 = acc0 + qvec * kv[0, d]
            acc1 = acc1 + qvec * kv[1, d]
            acc2 = acc2 + qvec * kv[2, d]
        s0 = acc0 * iknv[0]
        s1 = acc1 * iknv[1]
        s2 = acc2 * iknv[2]
        zero = jnp.zeros((_L,), jnp.int32)
        pid = (
            jnp.where(s0 >= s1, jnp.int32(4), zero)
            + jnp.where(s0 >= s2, jnp.int32(2), zero)
            + jnp.where(s1 >= s2, jnp.int32(1), zero)
        )
        idxv[pl.ds(g * _L, _L)] = pid
        return carry

    lax.fori_loop(0, qw // _L, group, 0)

    nfull = _ROW // _L
    tail = _ROW - _L
    for ch in range(nch):
        def copyg(g, carry):
            pv = idxv[pl.ds(ch * _CHUNK + g * _L, _L)]
            for lane in range(_L):
                p = pv[lane]
                dst = (g * _L + lane) * _ROW
                for j in range(nfull):
                    chunkbuf[pl.ds(dst + j * _L, _L)] = tabv[p, pl.ds(j * _L, _L)]
                chunkbuf[pl.ds(dst + tail, _L)] = tabv[p, pl.ds(tail, _L)]
            return carry

        lax.fori_loop(0, _CHUNK // _L, copyg, 0)
        pltpu.sync_copy(
            chunkbuf,
            out_hbm.at[pl.ds((qbase + ch * _CHUNK) * _ROW, _CHUNK * _ROW)],
        )


def kernel(queries, mem_keys, mem_values, top_num):
    bsz, dq = queries.shape
    m = mem_keys.shape[0]
    del top_num
    assert (m, dq) == (3, _DQ)
    qw = bsz // _NW
    nch = qw // _CHUNK

    qb = _round_to_bf16(queries)
    kb = _round_to_bf16(mem_keys)
    knorm = jnp.maximum(jnp.linalg.norm(mem_keys, axis=1), EPS)
    ksplat = jnp.broadcast_to(kb[:, :, None], (m, dq, _L))
    iknsplat = jnp.broadcast_to((1.0 / knorm).astype(jnp.float32)[:, None], (m, _L))
    table = jnp.stack(
        [mem_values[jnp.array(p)].reshape(-1) for p in _PERMS]
    ).astype(jnp.float32)

    mesh = plsc.VectorSubcoreMesh(core_axis_name="c", subcore_axis_name="s")
    run = pl.kernel(
        functools.partial(_sc_body, qw=qw, nch=nch),
        out_type=jax.ShapeDtypeStruct((bsz * _ROW,), jnp.float32),
        mesh=mesh,
        scratch_types=[
            pltpu.VMEM((_DQ, qw), jnp.float32),
            pltpu.VMEM((m, dq, _L), jnp.float32),
            pltpu.VMEM((m, _L), jnp.float32),
            pltpu.VMEM((8, _ROW), jnp.float32),
            pltpu.VMEM((qw,), jnp.int32),
            pltpu.VMEM((_CHUNK * _ROW,), jnp.float32),
        ],
    )
    out = run(qb.T, ksplat, iknsplat, table)
    return out.reshape(bsz, m, mem_values.shape[1], mem_values.shape[2])
